# Initial kernel scaffold; baseline (speedup 1.0000x reference)
import functools

import jax
import jax.numpy as jnp
from jax import lax
from jax.experimental import pallas as pl
from jax.experimental.pallas import tpu as pltpu
from jax.experimental.pallas import tpu_sc as plsc

VOCAB = 1000000
EMBED = 64
SEQ = 200
BATCH = 1024

NC = 2
NS = 16
L = 16
NW = NC * NS
TOK = BATCH * SEQ
PER_W = TOK // NW
IW = 128
K = 5
C = K * IW
NCHUNK = PER_W // C


def _prep_body(seg_tab_ref, pos_ref, seg_lab_ref, combo_ref, cidx_ref):
    combo = seg_tab_ref[:][:, None, :] + pos_ref[:][None, :, :]
    combo_ref[:] = combo.reshape(5 * SEQ, EMBED)
    pos_ids = lax.broadcasted_iota(jnp.int32, (BATCH, SEQ), 1)
    cidx_ref[:] = seg_lab_ref[:] * SEQ + pos_ids


_prep = pl.pallas_call(
    _prep_body,
    out_shape=(
        jax.ShapeDtypeStruct((5 * SEQ, EMBED), jnp.float32),
        jax.ShapeDtypeStruct((BATCH, SEQ), jnp.int32),
    ),
)


def _sc_body(seq_hbm, cidx_hbm, tok_tab, combo_hbm, out_hbm,
             tidx_v, cidx_v, tok_v, cmb_v, sem_t, sem_c):
    wid = lax.axis_index("s") * NC + lax.axis_index("c")

    def chunk(i, carry):
        tok0 = wid * PER_W + i * C
        r0 = wid * (PER_W // IW) + i * K
        pltpu.sync_copy(seq_hbm.at[pl.ds(r0, K)], tidx_v)
        pltpu.sync_copy(cidx_hbm.at[pl.ds(r0, K)], cidx_v)
        descs = []
        for j in range(K):
            dst = pl.ds(j * IW, IW)
            descs.append(pltpu.async_copy(
                tok_tab.at[tidx_v.at[j]], tok_v.at[dst], sem_t))
            descs.append(pltpu.async_copy(
                combo_hbm.at[cidx_v.at[j]], cmb_v.at[dst], sem_c))
        for d in descs:
            d.wait()

        def add_row(r, c2):
            for j in range(EMBED // L):
                sl = pl.ds(j * L, L)
                tok_v[r, sl] = tok_v[r, sl] + cmb_v[r, sl]
            return c2

        lax.fori_loop(0, C, add_row, 0)
        pltpu.sync_copy(tok_v, out_hbm.at[pl.ds(tok0, C)])
        return carry

    lax.fori_loop(0, NCHUNK, chunk, 0)


_sc_embed = functools.partial(
    pl.kernel,
    out_type=jax.ShapeDtypeStruct((TOK, EMBED), jnp.float32),
    mesh=plsc.VectorSubcoreMesh(core_axis_name="c", subcore_axis_name="s"),
    scratch_types=[
        pltpu.VMEM((K, IW), jnp.int32),
        pltpu.VMEM((K, IW), jnp.int32),
        pltpu.VMEM((C, EMBED), jnp.float32),
        pltpu.VMEM((C, EMBED), jnp.float32),
        pltpu.SemaphoreType.DMA,
        pltpu.SemaphoreType.DMA,
    ],
)(_sc_body)


@jax.jit
def kernel(sequence, segment_label, token_table, seg_table, pos_embed):
    combo, cidx = _prep(seg_table, pos_embed, segment_label.astype(jnp.int32))
    seq2 = sequence.astype(jnp.int32).reshape(TOK // IW, IW)
    cidx2 = cidx.reshape(TOK // IW, IW)
    out = _sc_embed(seq2, cidx2, token_table, combo)
    return out.reshape(BATCH, SEQ, EMBED)


# speedup vs baseline: 1.2515x; 1.2515x over previous
import functools

import jax
import jax.numpy as jnp
from jax import lax
from jax.experimental import pallas as pl
from jax.experimental.pallas import tpu as pltpu
from jax.experimental.pallas import tpu_sc as plsc

VOCAB = 1000000
EMBED = 64
SEQ = 200
BATCH = 1024

NC = 2
NS = 16
L = 16
NW = NC * NS
TOK = BATCH * SEQ
PER_W = TOK // NW
IW = 128
K = 5
C = K * IW
NCHUNK = PER_W // C


def _prep_body(seg_tab_ref, pos_ref, seg_lab_ref, combo_ref, cidx_ref):
    combo = seg_tab_ref[:][:, None, :] + pos_ref[:][None, :, :]
    combo_ref[:] = combo.reshape(5 * SEQ, EMBED)
    pos_ids = lax.broadcasted_iota(jnp.int32, (BATCH, SEQ), 1)
    cidx_ref[:] = seg_lab_ref[:] * SEQ + pos_ids


_prep = pl.pallas_call(
    _prep_body,
    out_shape=(
        jax.ShapeDtypeStruct((5 * SEQ, EMBED), jnp.float32),
        jax.ShapeDtypeStruct((BATCH, SEQ), jnp.int32),
    ),
)


def _sc_body(seq_hbm, cidx_hbm, tok_tab, combo_hbm, out_hbm,
             tidx_v, cidx_v, tok_v, cmb_v, sem_t, sem_c):
    wid = lax.axis_index("s") * NC + lax.axis_index("c")

    def chunk(i, carry):
        tok0 = wid * PER_W + i * C
        pltpu.sync_copy(seq_hbm.at[pl.ds(tok0, C)], tidx_v)
        pltpu.sync_copy(cidx_hbm.at[pl.ds(tok0, C)], cidx_v)
        descs = []
        for j in range(K):
            sl = pl.ds(j * IW, IW)
            descs.append(pltpu.async_copy(
                tok_tab.at[tidx_v.at[sl]], tok_v.at[sl], sem_t))
            descs.append(pltpu.async_copy(
                combo_hbm.at[cidx_v.at[sl]], cmb_v.at[sl], sem_c))
        for d in descs:
            d.wait()

        def add_row(r, c2):
            for j in range(EMBED // L):
                sl = pl.ds(j * L, L)
                tok_v[r, sl] = tok_v[r, sl] + cmb_v[r, sl]
            return c2

        lax.fori_loop(0, C, add_row, 0)
        pltpu.sync_copy(tok_v, out_hbm.at[pl.ds(tok0, C)])
        return carry

    lax.fori_loop(0, NCHUNK, chunk, 0)


_sc_embed = functools.partial(
    pl.kernel,
    out_type=jax.ShapeDtypeStruct((TOK, EMBED), jnp.float32),
    mesh=plsc.VectorSubcoreMesh(core_axis_name="c", subcore_axis_name="s"),
    compiler_params=pltpu.CompilerParams(use_tc_tiling_on_sc=False),
    scratch_types=[
        pltpu.VMEM((C,), jnp.int32),
        pltpu.VMEM((C,), jnp.int32),
        pltpu.VMEM((C, EMBED), jnp.float32),
        pltpu.VMEM((C, EMBED), jnp.float32),
        pltpu.SemaphoreType.DMA,
        pltpu.SemaphoreType.DMA,
    ],
)(_sc_body)


@jax.jit
def kernel(sequence, segment_label, token_table, seg_table, pos_embed):
    combo, cidx = _prep(seg_table, pos_embed, segment_label.astype(jnp.int32))
    seq2 = sequence.astype(jnp.int32).reshape(TOK)
    cidx2 = cidx.reshape(TOK)
    out = _sc_embed(seq2, cidx2, token_table, combo)
    return out.reshape(BATCH, SEQ, EMBED)


# speedup vs baseline: 1.2526x; 1.0009x over previous
import functools

import jax
import jax.numpy as jnp
from jax import lax
from jax.experimental import pallas as pl
from jax.experimental.pallas import tpu as pltpu
from jax.experimental.pallas import tpu_sc as plsc

VOCAB = 1000000
EMBED = 64
SEQ = 200
BATCH = 1024

NC = 2
NS = 16
L = 16
NW = NC * NS
TOK = BATCH * SEQ
ROWS_W = BATCH // NW
RC = 4
C = RC * SEQ
NCHUNK = ROWS_W // RC
GATHER_SPLITS = ((0, 80), (80, 80), (160, 40))


def _prep_body(seg_tab_ref, pos_ref, seg_lab_ref, combo_ref, cidx_ref):
    combo = seg_tab_ref[:][:, None, :] + pos_ref[:][None, :, :]
    combo_ref[:] = combo.reshape(5 * SEQ, EMBED)
    pos_ids = lax.broadcasted_iota(jnp.int32, (BATCH, SEQ), 1)
    cidx_ref[:] = seg_lab_ref[:] * SEQ + pos_ids


_prep = pl.pallas_call(
    _prep_body,
    out_shape=(
        jax.ShapeDtypeStruct((5 * SEQ, EMBED), jnp.float32),
        jax.ShapeDtypeStruct((BATCH, SEQ), jnp.int32),
    ),
)


def _sc_body(seq_hbm, cidx_hbm, tok_tab, combo_hbm, out_hbm,
             tidx_v, cidx_v, tok_v, cmb_v, sem_t, sem_c):
    wid = lax.axis_index("s") * NC + lax.axis_index("c")

    def chunk(i, carry):
        row0 = wid * ROWS_W + i * RC
        tok0 = row0 * SEQ
        pltpu.sync_copy(seq_hbm.at[pl.ds(tok0, C)], tidx_v)
        pltpu.sync_copy(cidx_hbm.at[pl.ds(tok0, C)], cidx_v)
        descs = []
        for b in range(RC):
            for l0, sz in GATHER_SPLITS:
                isl = pl.ds(b * SEQ + l0, sz)
                dsl = pl.ds(l0, sz)
                descs.append(pltpu.async_copy(
                    tok_tab.at[tidx_v.at[isl]], tok_v.at[b, dsl], sem_t))
                descs.append(pltpu.async_copy(
                    combo_hbm.at[cidx_v.at[isl]], cmb_v.at[b, dsl], sem_c))
        for d in descs:
            d.wait()

        def add_row(l, c2):
            for b in range(RC):
                for j in range(EMBED // L):
                    sl = pl.ds(j * L, L)
                    tok_v[b, l, sl] = tok_v[b, l, sl] + cmb_v[b, l, sl]
            return c2

        lax.fori_loop(0, SEQ, add_row, 0)
        pltpu.sync_copy(tok_v, out_hbm.at[pl.ds(row0, RC)])
        return carry

    lax.fori_loop(0, NCHUNK, chunk, 0)


_sc_embed = functools.partial(
    pl.kernel,
    out_type=jax.ShapeDtypeStruct((BATCH, SEQ, EMBED), jnp.float32),
    mesh=plsc.VectorSubcoreMesh(core_axis_name="c", subcore_axis_name="s"),
    compiler_params=pltpu.CompilerParams(use_tc_tiling_on_sc=False),
    scratch_types=[
        pltpu.VMEM((C,), jnp.int32),
        pltpu.VMEM((C,), jnp.int32),
        pltpu.VMEM((RC, SEQ, EMBED), jnp.float32),
        pltpu.VMEM((RC, SEQ, EMBED), jnp.float32),
        pltpu.SemaphoreType.DMA,
        pltpu.SemaphoreType.DMA,
    ],
)(_sc_body)


@jax.jit
def kernel(sequence, segment_label, token_table, seg_table, pos_embed):
    combo, cidx = _prep(seg_table, pos_embed, segment_label.astype(jnp.int32))
    seq1 = sequence.astype(jnp.int32).reshape(TOK)
    cidx1 = cidx.reshape(TOK)
    return _sc_embed(seq1, cidx1, token_table, combo)
